# baseline (speedup 1.0000x reference)
import jax
import jax.numpy as jnp
from jax import lax
from jax.experimental import pallas as pl
from jax.experimental.pallas import tpu as pltpu
from jax.experimental.pallas import tpu_sc as plsc

M_ITEM = 1_000_000
DIM = 16
N_ROWS = 819_200
LR = 0.01

LANES = 16
C = 4_096
NFULL = M_ITEM // C
TAIL = M_ITEM - NFULL * C
NW = 32
B = 512
NEDGE = NFULL + 2


def _bval(bounds2, u):
    lane = lax.broadcasted_iota(jnp.int32, (LANES,), 0)
    row = bounds2[u // LANES]
    return jnp.sum(jnp.where(lane == (u % LANES), row, 0))


def _body(emb_t, items, grads4, bounds, out_t, bounds2, chunk, kv, gblk):
    c = lax.axis_index("c")
    sid = lax.axis_index("s")
    w = sid * 2 + c
    pltpu.sync_copy(bounds, bounds2)

    def do_unit(u, i0, width, lo, hi):
        pltpu.sync_copy(emb_t.at[:, pl.ds(i0, width)],
                        chunk.at[:, pl.ds(0, width)])

        lo_a = (lo // 128) * 128
        nblk = (hi - lo_a + (B - 1)) // B

        def blk(b, carry):
            s_nom = lo_a + b * B
            s = pl.multiple_of(jnp.minimum(s_nom, N_ROWS - B), 128)
            pltpu.sync_copy(items.at[pl.ds(s, B)], kv)
            pltpu.sync_copy(grads4.at[:, pl.ds(s // 128, B // 128)], gblk)
            glo = jnp.maximum(lo, s_nom)
            for q in range(B // LANES):
                keys = kv[pl.ds(q * LANES, LANES)]
                g = s + q * LANES + lax.broadcasted_iota(jnp.int32, (LANES,), 0)
                valid = (g >= glo) & (g < hi)
                idx = jnp.where(valid, keys - i0, 0)
                for d in range(DIM):
                    v = gblk[d // 8, q // 8, d % 8,
                             pl.ds((q % 8) * LANES, LANES)] * (-LR)
                    plsc.addupdate_scatter(chunk.at[d], [idx], v, mask=valid)
            return carry
        lax.fori_loop(0, nblk, blk, 0)

        pltpu.sync_copy(chunk.at[:, pl.ds(0, width)],
                        out_t.at[:, pl.ds(i0, width)])

    bounds2v = bounds2

    my_units = (NFULL - w + (NW - 1)) // NW

    def unit_body(k, carry):
        u = w + k * NW
        i0 = pl.multiple_of(u * C, 128)
        lo = _bval(bounds2v, u)
        hi = _bval(bounds2v, u + 1)
        do_unit(u, i0, C, lo, hi)
        return carry
    lax.fori_loop(0, my_units, unit_body, 0)

    @pl.when(w == NW - 1)
    def _():
        lo = _bval(bounds2v, NFULL)
        hi = _bval(bounds2v, NFULL + 1)
        do_unit(NFULL, NFULL * C, TAIL, lo, hi)


def kernel(items_emb, items, items_emb_grad):
    items = items.astype(jnp.int32)
    edges = jnp.arange(0, NEDGE, dtype=jnp.int32) * C
    edges = jnp.minimum(edges, M_ITEM)
    bounds = jnp.searchsorted(items, edges, side="left").astype(jnp.int32)
    bounds = jnp.pad(bounds, (0, 256 - NEDGE)).reshape(16, 16)

    mesh = plsc.VectorSubcoreMesh(core_axis_name="c", subcore_axis_name="s")
    run = pl.kernel(
        _body,
        out_type=jax.ShapeDtypeStruct((DIM, M_ITEM), jnp.float32),
        mesh=mesh,
        scratch_types=[
            pltpu.VMEM((16, 16), jnp.int32),
            pltpu.VMEM((DIM, C), jnp.float32),
            pltpu.VMEM((B,), jnp.int32),
            pltpu.VMEM((2, B // 128, 8, 128), jnp.float32),
        ],
        compiler_params=pltpu.CompilerParams(
            use_tc_tiling_on_sc=False, needs_layout_passes=False),
    )
    grads4 = (items_emb_grad.T.reshape(2, 8, N_ROWS // 128, 128)
              .transpose(0, 2, 1, 3))
    out_t = run(items_emb.T, items, grads4, bounds)
    return out_t.T


# speedup vs baseline: 1.0289x; 1.0289x over previous
import jax
import jax.numpy as jnp
from jax import lax
from jax.experimental import pallas as pl
from jax.experimental.pallas import tpu as pltpu
from jax.experimental.pallas import tpu_sc as plsc

M_ITEM = 1_000_000
DIM = 16
N_ROWS = 819_200
LR = 0.01

LANES = 16
C = 4_096
NFULL = M_ITEM // C
TAIL = M_ITEM - NFULL * C
NW = 32
B = 512
NEDGE = NFULL + 2


def _bval(bounds2, u):
    lane = lax.broadcasted_iota(jnp.int32, (LANES,), 0)
    row = bounds2[u // LANES]
    return jnp.sum(jnp.where(lane == (u % LANES), row, 0))


def _body(emb_t, items, grads4, bounds, out_t, bounds2, chunk, kv, gblk):
    c = lax.axis_index("c")
    sid = lax.axis_index("s")
    w = sid * 2 + c
    pltpu.sync_copy(bounds, bounds2)

    def do_unit(u, i0, width, lo, hi):
        pltpu.sync_copy(emb_t.at[:, pl.ds(i0, width)],
                        chunk.at[:, pl.ds(0, width)])

        lo_a = (lo // 128) * 128
        nblk = (hi - lo_a + (B - 1)) // B

        def blk(b, carry):
            s_nom = lo_a + b * B
            s = pl.multiple_of(jnp.minimum(s_nom, N_ROWS - B), 128)
            pltpu.sync_copy(items.at[pl.ds(s, B)], kv)
            pltpu.sync_copy(grads4.at[:, pl.ds(s // 128, B // 128)], gblk)
            glo = jnp.maximum(lo, s_nom)
            for q in range(B // LANES):
                keys = kv[pl.ds(q * LANES, LANES)]
                g = s + q * LANES + lax.broadcasted_iota(jnp.int32, (LANES,), 0)
                valid = (g >= glo) & (g < hi)
                idx = jnp.where(valid, keys - i0, 0)
                for d in range(DIM):
                    v = gblk[d // 8, q // 8, d % 8,
                             pl.ds((q % 8) * LANES, LANES)] * (-LR)
                    plsc.addupdate_scatter(chunk.at[d], [idx], v, mask=valid)
            return carry
        lax.fori_loop(0, nblk, blk, 0)

        pltpu.sync_copy(chunk.at[:, pl.ds(0, width)],
                        out_t.at[:, pl.ds(i0, width)])

    bounds2v = bounds2

    my_units = (NFULL - w + (NW - 1)) // NW

    def unit_body(k, carry):
        u = w + k * NW
        i0 = pl.multiple_of(u * C, 128)
        lo = _bval(bounds2v, u)
        hi = _bval(bounds2v, u + 1)
        do_unit(u, i0, C, lo, hi)
        return carry
    lax.fori_loop(0, my_units, unit_body, 0)

    @pl.when(w == NW - 1)
    def _():
        lo = _bval(bounds2v, NFULL)
        hi = _bval(bounds2v, NFULL + 1)
        do_unit(NFULL, NFULL * C, TAIL, lo, hi)


def kernel(items_emb, items, items_emb_grad):
    items = items.astype(jnp.int32)
    edges = jnp.arange(0, NEDGE, dtype=jnp.int32) * C
    edges = jnp.minimum(edges, M_ITEM)
    items2d = items.reshape(N_ROWS // 128, 128)
    coarse = items2d[:, 0]
    cj = jnp.sum((coarse[None, :] < edges[:, None]).astype(jnp.int32), axis=1)
    jj = jnp.maximum(cj - 1, 0)
    rows = jnp.take(items2d, jj, axis=0)
    inner = jnp.sum((rows < edges[:, None]).astype(jnp.int32), axis=1)
    bounds = jnp.where(cj == 0, 0, jj * 128 + inner).astype(jnp.int32)
    bounds = jnp.pad(bounds, (0, 256 - NEDGE)).reshape(16, 16)

    mesh = plsc.VectorSubcoreMesh(core_axis_name="c", subcore_axis_name="s")
    run = pl.kernel(
        _body,
        out_type=jax.ShapeDtypeStruct((DIM, M_ITEM), jnp.float32),
        mesh=mesh,
        scratch_types=[
            pltpu.VMEM((16, 16), jnp.int32),
            pltpu.VMEM((DIM, C), jnp.float32),
            pltpu.VMEM((B,), jnp.int32),
            pltpu.VMEM((2, B // 128, 8, 128), jnp.float32),
        ],
        compiler_params=pltpu.CompilerParams(
            use_tc_tiling_on_sc=False, needs_layout_passes=False),
    )
    grads4 = (items_emb_grad.T.reshape(2, 8, N_ROWS // 128, 128)
              .transpose(0, 2, 1, 3))
    out_t = run(items_emb.T, items, grads4, bounds)
    return out_t.T
